# Initial kernel scaffold; baseline (speedup 1.0000x reference)
import functools

import jax
import jax.numpy as jnp
from jax import lax
from jax.experimental import pallas as pl
from jax.experimental.pallas import tpu as pltpu
from jax.experimental.pallas import tpu_sc as plsc

N = 10000
E = 320000
D = 128
NPAD = 10240
NW = 32
EPW = E // NW
K = 80
NCH = EPW // K

_mesh = plsc.VectorSubcoreMesh(core_axis_name="c", subcore_axis_name="s")


def _wid():
    return lax.axis_index("s") * 2 + lax.axis_index("c")


@functools.partial(
    pl.kernel, mesh=_mesh,
    out_type=jax.ShapeDtypeStruct((2, NPAD), jnp.float32),
    scratch_types=[
        pltpu.VMEM((K,), jnp.int32),
        pltpu.VMEM((K,), jnp.float32),
        pltpu.VMEM((NPAD // 16,), jnp.float32),
        pltpu.VMEM_SHARED((NPAD,), jnp.float32),
    ],
)
def _deg(col_hbm, ew_hbm, out_hbm, colv, ewv, zv, acc):
    c = lax.axis_index("c")
    s = lax.axis_index("s")
    w = _wid()
    nz = NPAD // 16

    def zfill(r, _):
        zv[pl.ds(r * 16, 16)] = jnp.zeros((16,), jnp.float32)
        return 0

    lax.fori_loop(0, nz // 16, zfill, 0)
    pltpu.sync_copy(zv, acc.at[pl.ds(s * nz, nz)])
    plsc.subcore_barrier()

    def body(g, _):
        base = w * EPW + g * K
        pltpu.sync_copy(col_hbm.at[pl.ds(base, K)], colv)
        pltpu.sync_copy(ew_hbm.at[pl.ds(base, K)], ewv)
        pltpu.sync_copy(ewv, acc.at[colv], add=True)
        return 0

    lax.fori_loop(0, NCH, body, 0)
    plsc.subcore_barrier()
    pltpu.sync_copy(acc.at[pl.ds(s * nz, nz)], out_hbm.at[c, pl.ds(s * nz, nz)])


@functools.partial(
    pl.kernel, mesh=_mesh,
    out_type=jax.ShapeDtypeStruct((E,), jnp.float32),
    scratch_types=[
        pltpu.VMEM((K,), jnp.int32),
        pltpu.VMEM((K,), jnp.int32),
        pltpu.VMEM((K,), jnp.float32),
        pltpu.VMEM((K,), jnp.float32),
        pltpu.VMEM((K,), jnp.float32),
        pltpu.VMEM((K,), jnp.float32),
        pltpu.SemaphoreType.DMA,
    ],
)
def _norm(row_hbm, col_hbm, ew_hbm, dinv_hbm, nrm_hbm, rv, cv, ewv, drv, dcv,
          nv, sem):
    w = _wid()

    def body(g, _):
        base = w * EPW + g * K
        pltpu.sync_copy(row_hbm.at[pl.ds(base, K)], rv)
        pltpu.sync_copy(col_hbm.at[pl.ds(base, K)], cv)
        pltpu.sync_copy(ew_hbm.at[pl.ds(base, K)], ewv)
        pltpu.async_copy(dinv_hbm.at[rv], drv, sem).wait()
        pltpu.async_copy(dinv_hbm.at[cv], dcv, sem).wait()
        for j in range(K // 16):
            sl = pl.ds(j * 16, 16)
            nv[sl] = drv[sl] * ewv[sl] * dcv[sl]
        pltpu.sync_copy(nv, nrm_hbm.at[pl.ds(base, K)])
        return 0

    lax.fori_loop(0, NCH, body, 0)


@functools.partial(
    pl.kernel, mesh=_mesh,
    out_type=jax.ShapeDtypeStruct((2, N, D), jnp.float32),
    scratch_types=[
        pltpu.VMEM((K,), jnp.int32),
        pltpu.VMEM((K,), jnp.int32),
        pltpu.VMEM((K,), jnp.float32),
        pltpu.VMEM((K, D), jnp.float32),
        pltpu.VMEM((125, D), jnp.float32),
        pltpu.VMEM_SHARED((N, D), jnp.float32),
        pltpu.SemaphoreType.DMA,
    ],
)
def _spmm(h_hbm, row_hbm, col_hbm, nrm_hbm, out_hbm, rv, cv, nv, rows, zrows,
          acc, sem):
    c = lax.axis_index("c")
    s = lax.axis_index("s")
    w = _wid()
    npr = N // 16

    def zfill(r, _):
        for d in range(D // 16):
            zrows[r, pl.ds(d * 16, 16)] = jnp.zeros((16,), jnp.float32)
        return 0

    lax.fori_loop(0, 125, zfill, 0)
    for j in range(npr // 125):
        pltpu.sync_copy(zrows, acc.at[pl.ds(s * npr + j * 125, 125)])
    plsc.subcore_barrier()

    def body(g, _):
        base = w * EPW + g * K
        pltpu.sync_copy(row_hbm.at[pl.ds(base, K)], rv)
        pltpu.sync_copy(col_hbm.at[pl.ds(base, K)], cv)
        pltpu.sync_copy(nrm_hbm.at[pl.ds(base, K)], nv)
        pltpu.async_copy(h_hbm.at[rv], rows, sem).wait()

        def scale(k, _):
            sc = nv[k]
            for d in range(D // 16):
                sl = pl.ds(d * 16, 16)
                rows[k, sl] = rows[k, sl] * sc
            return 0

        lax.fori_loop(0, K, scale, 0)
        pltpu.sync_copy(rows, acc.at[cv], add=True)
        return 0

    lax.fori_loop(0, NCH, body, 0)
    plsc.subcore_barrier()
    pltpu.sync_copy(acc.at[pl.ds(s * npr, npr)],
                    out_hbm.at[c, pl.ds(s * npr, npr)])


_RB = 1250


def _dinv_body(dref, dinv_ref, dinv2_ref):
    v = dref[...]
    deg = 1.0 + v[:NPAD // D] + v[NPAD // D:]
    r = jnp.where(deg > 0, lax.rsqrt(jnp.maximum(deg, 1e-30)), 0.0)
    dinv_ref[...] = r
    dinv2_ref[...] = r * r


def _mm_body(xb, wb, ob):
    ob[...] = jnp.dot(xb[...], wb[...], preferred_element_type=jnp.float32)


def _merge_mm_body(p0, p1, hb, di2, bb, wb, ob):
    t = p0[0] + p1[0] + hb[...] * di2[...] + bb[...]
    a = jnp.where(t > 0, t, jnp.expm1(t))
    ob[...] = jnp.dot(a, wb[...], preferred_element_type=jnp.float32)


def _final_body(p0, p1, hb, di2, bb, wb, blb, ob):
    t = p0[0] + p1[0] + hb[...] * di2[...] + bb[...]
    a = jnp.where(t > 0, t, jnp.expm1(t))
    y = jnp.dot(a, wb[...], preferred_element_type=jnp.float32) + blb[...]
    ob[...] = jnp.maximum(y, 0.0)


def _mm(x, W):
    return pl.pallas_call(
        _mm_body,
        grid=(N // _RB,),
        in_specs=[pl.BlockSpec((_RB, D), lambda i: (i, 0)),
                  pl.BlockSpec((D, D), lambda i: (0, 0))],
        out_specs=pl.BlockSpec((_RB, D), lambda i: (i, 0)),
        out_shape=jax.ShapeDtypeStruct((N, D), jnp.float32),
    )(x, W)


def _merge_mm(p, h, di2, b, W):
    return pl.pallas_call(
        _merge_mm_body,
        grid=(N // _RB,),
        in_specs=[pl.BlockSpec((1, _RB, D), lambda i: (0, i, 0)),
                  pl.BlockSpec((1, _RB, D), lambda i: (1, i, 0)),
                  pl.BlockSpec((_RB, D), lambda i: (i, 0)),
                  pl.BlockSpec((_RB, 1), lambda i: (i, 0)),
                  pl.BlockSpec((1, D), lambda i: (0, 0)),
                  pl.BlockSpec((D, D), lambda i: (0, 0))],
        out_specs=pl.BlockSpec((_RB, D), lambda i: (i, 0)),
        out_shape=jax.ShapeDtypeStruct((N, D), jnp.float32),
    )(p, p, h, di2, b, W)


def _final(p, h, di2, b, W, bl):
    return pl.pallas_call(
        _final_body,
        grid=(N // _RB,),
        in_specs=[pl.BlockSpec((1, _RB, D), lambda i: (0, i, 0)),
                  pl.BlockSpec((1, _RB, D), lambda i: (1, i, 0)),
                  pl.BlockSpec((_RB, D), lambda i: (i, 0)),
                  pl.BlockSpec((_RB, 1), lambda i: (i, 0)),
                  pl.BlockSpec((1, D), lambda i: (0, 0)),
                  pl.BlockSpec((D, D), lambda i: (0, 0)),
                  pl.BlockSpec((1, D), lambda i: (0, 0))],
        out_specs=pl.BlockSpec((_RB, D), lambda i: (i, 0)),
        out_shape=jax.ShapeDtypeStruct((N, D), jnp.float32),
    )(p, p, h, di2, b, W, bl)


def _dinv_call(degp):
    return pl.pallas_call(
        _dinv_body,
        out_shape=(jax.ShapeDtypeStruct((NPAD // D, D), jnp.float32),
                   jax.ShapeDtypeStruct((NPAD // D, D), jnp.float32)),
    )(degp)


def kernel(x, edge_index, edge_feats, W1, b1, W2, b2, W3, b3, Wlin, blin):
    row = edge_index[0]
    col = edge_index[1]
    ew = edge_feats

    degp = _deg(col, ew)
    dinv2d, dinv2_2d = _dinv_call(degp.reshape(2 * NPAD // D, D))
    dinv_flat = dinv2d.reshape(NPAD)
    di2 = dinv2_2d.reshape(NPAD)[:N].reshape(N, 1)
    nrm = _norm(row, col, ew, dinv_flat)

    b1r, b2r, b3r = b1.reshape(1, D), b2.reshape(1, D), b3.reshape(1, D)
    blr = blin.reshape(1, D)

    h1 = _mm(x, W1)
    p1 = _spmm(h1, row, col, nrm)
    h2 = _merge_mm(p1, h1, di2, b1r, W2)
    p2 = _spmm(h2, row, col, nrm)
    h3 = _merge_mm(p2, h2, di2, b2r, W3)
    p3 = _spmm(h3, row, col, nrm)
    return _final(p3, h3, di2, b3r, Wlin, blr)


# speedup vs baseline: 6.8930x; 6.8930x over previous
import functools

import jax
import jax.numpy as jnp
from jax import lax
from jax.experimental import pallas as pl
from jax.experimental.pallas import tpu as pltpu
from jax.experimental.pallas import tpu_sc as plsc

N = 10000
E = 320000
D = 128
NPAD = 10240
NW = 32
EPW = E // NW
K = 80
NCH = EPW // K

_mesh = plsc.VectorSubcoreMesh(core_axis_name="c", subcore_axis_name="s")


def _wid():
    return lax.axis_index("s") * 2 + lax.axis_index("c")


@functools.partial(
    pl.kernel, mesh=_mesh,
    out_type=jax.ShapeDtypeStruct((2, NPAD), jnp.float32),
    scratch_types=[
        pltpu.VMEM((K,), jnp.int32),
        pltpu.VMEM((K,), jnp.float32),
        pltpu.VMEM((NPAD // 16,), jnp.float32),
        pltpu.VMEM_SHARED((NPAD,), jnp.float32),
    ],
)
def _deg(col_hbm, ew_hbm, out_hbm, colv, ewv, zv, acc):
    c = lax.axis_index("c")
    s = lax.axis_index("s")
    w = _wid()
    nz = NPAD // 16

    def zfill(r, _):
        zv[pl.ds(r * 16, 16)] = jnp.zeros((16,), jnp.float32)
        return 0

    lax.fori_loop(0, nz // 16, zfill, 0)
    pltpu.sync_copy(zv, acc.at[pl.ds(s * nz, nz)])
    plsc.subcore_barrier()

    def body(g, _):
        base = w * EPW + g * K
        pltpu.sync_copy(col_hbm.at[pl.ds(base, K)], colv)
        pltpu.sync_copy(ew_hbm.at[pl.ds(base, K)], ewv)
        pltpu.sync_copy(ewv, acc.at[colv], add=True)
        return 0

    lax.fori_loop(0, NCH, body, 0)
    plsc.subcore_barrier()
    pltpu.sync_copy(acc.at[pl.ds(s * nz, nz)], out_hbm.at[c, pl.ds(s * nz, nz)])


@functools.partial(
    pl.kernel, mesh=_mesh,
    out_type=jax.ShapeDtypeStruct((E,), jnp.float32),
    scratch_types=[
        pltpu.VMEM((K,), jnp.int32),
        pltpu.VMEM((K,), jnp.int32),
        pltpu.VMEM((K,), jnp.float32),
        pltpu.VMEM((K,), jnp.float32),
        pltpu.VMEM((K,), jnp.float32),
        pltpu.VMEM((K,), jnp.float32),
        pltpu.SemaphoreType.DMA,
    ],
)
def _norm(row_hbm, col_hbm, ew_hbm, dinv_hbm, nrm_hbm, rv, cv, ewv, drv, dcv,
          nv, sem):
    w = _wid()

    def body(g, _):
        base = w * EPW + g * K
        pltpu.sync_copy(row_hbm.at[pl.ds(base, K)], rv)
        pltpu.sync_copy(col_hbm.at[pl.ds(base, K)], cv)
        pltpu.sync_copy(ew_hbm.at[pl.ds(base, K)], ewv)
        pltpu.async_copy(dinv_hbm.at[rv], drv, sem).wait()
        pltpu.async_copy(dinv_hbm.at[cv], dcv, sem).wait()
        for j in range(K // 16):
            sl = pl.ds(j * 16, 16)
            nv[sl] = drv[sl] * ewv[sl] * dcv[sl]
        pltpu.sync_copy(nv, nrm_hbm.at[pl.ds(base, K)])
        return 0

    lax.fori_loop(0, NCH, body, 0)


@functools.partial(
    pl.kernel, mesh=_mesh,
    out_type=jax.ShapeDtypeStruct((2, NPAD, D), jnp.float32),
    scratch_types=[
        pltpu.VMEM((K,), jnp.int32),
        pltpu.VMEM((K,), jnp.int32),
        pltpu.VMEM((K,), jnp.float32),
        pltpu.VMEM((K, D), jnp.float32),
        pltpu.VMEM((128, D), jnp.float32),
        pltpu.VMEM_SHARED((NPAD, D), jnp.float32),
        pltpu.SemaphoreType.DMA,
    ],
)
def _spmm(h_hbm, row_hbm, col_hbm, nrm_hbm, out_hbm, rv, cv, nv, rows, zrows,
          acc, sem):
    c = lax.axis_index("c")
    s = lax.axis_index("s")
    w = _wid()
    npr = NPAD // 16

    def zfill(r, _):
        for d in range(D // 16):
            zrows[r, pl.ds(d * 16, 16)] = jnp.zeros((16,), jnp.float32)
        return 0

    lax.fori_loop(0, 128, zfill, 0)
    for j in range(npr // 128):
        pltpu.sync_copy(zrows, acc.at[pl.ds(s * npr + j * 128, 128)])
    plsc.subcore_barrier()

    def body(g, _):
        base = w * EPW + g * K
        pltpu.sync_copy(row_hbm.at[pl.ds(base, K)], rv)
        pltpu.sync_copy(col_hbm.at[pl.ds(base, K)], cv)
        pltpu.sync_copy(nrm_hbm.at[pl.ds(base, K)], nv)
        pltpu.async_copy(h_hbm.at[rv], rows, sem).wait()

        def scale(g16, _):
            nvec = nv[pl.ds(g16 * 16, 16)]
            for j in range(16):
                sc = nvec[j]
                k = g16 * 16 + j
                for d in range(D // 16):
                    sl = pl.ds(d * 16, 16)
                    rows[k, sl] = rows[k, sl] * sc
            return 0

        lax.fori_loop(0, K // 16, scale, 0)
        pltpu.sync_copy(rows, acc.at[cv], add=True)
        return 0

    lax.fori_loop(0, NCH, body, 0)
    plsc.subcore_barrier()
    pltpu.sync_copy(acc.at[pl.ds(s * npr, npr)],
                    out_hbm.at[c, pl.ds(s * npr, npr)])


_RB = 1000


def _dinv_body(dref, dinv_ref, dinv2_ref):
    v = dref[...]
    deg = 1.0 + v[:NPAD // D] + v[NPAD // D:]
    r = jnp.where(deg > 0, lax.rsqrt(jnp.maximum(deg, 1e-30)), 0.0)
    dinv_ref[...] = r
    dinv2_ref[...] = r * r


def _mm_body(xb, wb, ob):
    ob[...] = jnp.dot(xb[...], wb[...], preferred_element_type=jnp.float32)


def _merge_mm_body(p0, p1, hb, di2, bb, wb, ob):
    t = p0[0] + p1[0] + hb[...] * di2[...] + bb[...]
    a = jnp.where(t > 0, t, (jnp.exp(t) - 1.0))
    ob[...] = jnp.dot(a, wb[...], preferred_element_type=jnp.float32)


def _final_body(p0, p1, hb, di2, bb, wb, blb, ob):
    t = p0[0] + p1[0] + hb[...] * di2[...] + bb[...]
    a = jnp.where(t > 0, t, (jnp.exp(t) - 1.0))
    y = jnp.dot(a, wb[...], preferred_element_type=jnp.float32) + blb[...]
    ob[...] = jnp.maximum(y, 0.0)


def _mm(x, W):
    return pl.pallas_call(
        _mm_body,
        grid=(N // _RB,),
        in_specs=[pl.BlockSpec((_RB, D), lambda i: (i, 0)),
                  pl.BlockSpec((D, D), lambda i: (0, 0))],
        out_specs=pl.BlockSpec((_RB, D), lambda i: (i, 0)),
        out_shape=jax.ShapeDtypeStruct((N, D), jnp.float32),
    )(x, W)


def _merge_mm(p, h, di2, b, W):
    return pl.pallas_call(
        _merge_mm_body,
        grid=(N // _RB,),
        in_specs=[pl.BlockSpec((1, _RB, D), lambda i: (0, i, 0)),
                  pl.BlockSpec((1, _RB, D), lambda i: (1, i, 0)),
                  pl.BlockSpec((_RB, D), lambda i: (i, 0)),
                  pl.BlockSpec((_RB, 1), lambda i: (i, 0)),
                  pl.BlockSpec((1, D), lambda i: (0, 0)),
                  pl.BlockSpec((D, D), lambda i: (0, 0))],
        out_specs=pl.BlockSpec((_RB, D), lambda i: (i, 0)),
        out_shape=jax.ShapeDtypeStruct((N, D), jnp.float32),
    )(p, p, h, di2, b, W)


def _final(p, h, di2, b, W, bl):
    return pl.pallas_call(
        _final_body,
        grid=(N // _RB,),
        in_specs=[pl.BlockSpec((1, _RB, D), lambda i: (0, i, 0)),
                  pl.BlockSpec((1, _RB, D), lambda i: (1, i, 0)),
                  pl.BlockSpec((_RB, D), lambda i: (i, 0)),
                  pl.BlockSpec((_RB, 1), lambda i: (i, 0)),
                  pl.BlockSpec((1, D), lambda i: (0, 0)),
                  pl.BlockSpec((D, D), lambda i: (0, 0)),
                  pl.BlockSpec((1, D), lambda i: (0, 0))],
        out_specs=pl.BlockSpec((_RB, D), lambda i: (i, 0)),
        out_shape=jax.ShapeDtypeStruct((N, D), jnp.float32),
    )(p, p, h, di2, b, W, bl)


def _dinv_call(degp):
    return pl.pallas_call(
        _dinv_body,
        out_shape=(jax.ShapeDtypeStruct((NPAD // D, D), jnp.float32),
                   jax.ShapeDtypeStruct((NPAD // D, D), jnp.float32)),
    )(degp)


def kernel(x, edge_index, edge_feats, W1, b1, W2, b2, W3, b3, Wlin, blin):
    row = edge_index[0]
    col = edge_index[1]
    ew = edge_feats

    degp = _deg(col, ew)
    dinv2d, dinv2_2d = _dinv_call(degp.reshape(2 * NPAD // D, D))
    dinv_flat = dinv2d.reshape(NPAD)
    di2 = dinv2_2d.reshape(NPAD)[:N].reshape(N, 1)
    nrm = _norm(row, col, ew, dinv_flat)

    b1r, b2r, b3r = b1.reshape(1, D), b2.reshape(1, D), b3.reshape(1, D)
    blr = blin.reshape(1, D)

    h1 = _mm(x, W1)
    p1 = _spmm(h1, row, col, nrm)
    h2 = _merge_mm(p1, h1, di2, b1r, W2)
    p2 = _spmm(h2, row, col, nrm)
    h3 = _merge_mm(p2, h2, di2, b2r, W3)
    p3 = _spmm(h3, row, col, nrm)
    return _final(p3, h3, di2, b3r, Wlin, blr)


# speedup vs baseline: 11.6172x; 1.6854x over previous
import functools

import jax
import jax.numpy as jnp
from jax import lax
from jax.experimental import pallas as pl
from jax.experimental.pallas import tpu as pltpu
from jax.experimental.pallas import tpu_sc as plsc

N = 10000
E = 320000
D = 128
HD = D // 2
NPAD = 10240
NW = 32
K = 80
NCH = E // NW // K
NB = 5
EPT = E // 16
NCHT = EPT // K
UNR = 10
NRB = 5

_mesh = plsc.VectorSubcoreMesh(core_axis_name="c", subcore_axis_name="s")


def _wid():
    return lax.axis_index("s") * 2 + lax.axis_index("c")


@functools.partial(
    pl.kernel, mesh=_mesh,
    out_type=jax.ShapeDtypeStruct((2, NPAD), jnp.float32),
    scratch_types=[
        pltpu.VMEM((NCH, K), jnp.int32),
        pltpu.VMEM((NCH, K), jnp.float32),
        pltpu.VMEM((NPAD // 16,), jnp.float32),
        pltpu.VMEM_SHARED((NPAD,), jnp.float32),
        pltpu.SemaphoreType.DMA,
    ],
)
def _deg(col2_hbm, ew2_hbm, out_hbm, cv2, ewv2, zv, acc, sem):
    c = lax.axis_index("c")
    s = lax.axis_index("s")
    nz = NPAD // 16

    def zfill(r, _):
        zv[pl.ds(r * 16, 16)] = jnp.zeros((16,), jnp.float32)
        return 0

    lax.fori_loop(0, nz // 16, zfill, 0)
    pltpu.sync_copy(zv, acc.at[pl.ds(s * nz, nz)])
    w = _wid()
    pltpu.sync_copy(col2_hbm.at[w], cv2)
    pltpu.sync_copy(ew2_hbm.at[w], ewv2)
    plsc.subcore_barrier()

    def round_(r, _):
        for b in range(NB):
            g = r * NB + b
            pltpu.async_copy(ewv2.at[g], acc.at[cv2.at[g]], sem, add=True)
        for b in range(NB):
            g = r * NB + b
            pltpu.make_async_copy(ewv2.at[g], acc.at[cv2.at[g]], sem).wait()
        return 0

    lax.fori_loop(0, NCH // NB, round_, 0)
    plsc.subcore_barrier()
    pltpu.sync_copy(acc.at[pl.ds(s * nz, nz)], out_hbm.at[c, pl.ds(s * nz, nz)])


@functools.partial(
    pl.kernel, mesh=_mesh,
    compiler_params=pltpu.CompilerParams(use_tc_tiling_on_sc=False),
    out_type=jax.ShapeDtypeStruct((2, NPAD, HD), jnp.float32),
    scratch_types=[
        pltpu.VMEM((UNR, 2, K), jnp.int32),
        pltpu.VMEM((UNR, K), jnp.float32),
        pltpu.VMEM((NRB, K, HD), jnp.float32),
        pltpu.VMEM_SHARED((NPAD, HD), jnp.float32),
        pltpu.SemaphoreType.DMA((UNR,)),
        pltpu.SemaphoreType.DMA((NRB,)),
        pltpu.SemaphoreType.DMA((NRB,)),
    ],
)
def _spmm(hst_hbm, edata_hbm, ew3_hbm, out_hbm, ebufs, ewbufs, rows, acc,
          isem, gsem, ssem):
    c = lax.axis_index("c")
    s = lax.axis_index("s")
    npr = NPAD // 16

    def zfill(r, _):
        for d in range(HD // 16):
            rows[0, r, pl.ds(d * 16, 16)] = jnp.zeros((16,), jnp.float32)
        return 0

    lax.fori_loop(0, K, zfill, 0)
    for j in range(npr // K):
        pltpu.sync_copy(rows.at[0], acc.at[pl.ds(s * npr + j * K, K)])
    plsc.subcore_barrier()

    def idx_load(g, slot):
        return (pltpu.make_async_copy(edata_hbm.at[s, g], ebufs.at[slot],
                                      isem.at[slot]),
                pltpu.make_async_copy(ew3_hbm.at[s, g], ewbufs.at[slot],
                                      isem.at[slot]))

    def idx_start(g, slot):
        a, b = idx_load(g, slot)
        a.start()
        b.start()

    def idx_wait(g, slot):
        a, b = idx_load(g, slot)
        a.wait()
        b.wait()

    def gather(g, slot, rslot):
        src = hst_hbm.at[c].at[ebufs.at[slot, 0]]
        return pltpu.make_async_copy(src, rows.at[rslot], gsem.at[rslot])

    def scatter(slot, rslot):
        return pltpu.make_async_copy(rows.at[rslot],
                                     acc.at[ebufs.at[slot, 1]],
                                     ssem.at[rslot])

    for g in range(4):
        idx_start(g, g)
    for g in range(2):
        idx_wait(g, g)
        gather(g, g, g).start()

    def round_(r, _):
        for u in range(UNR):
            g = r * UNR + u
            b5 = u % NRB
            bi2 = (u + 2) % UNR
            br2 = (u + 2) % NRB

            @pl.when(g + 4 < NCHT)
            def _issue_idx():
                idx_start(g + 4, (u + 4) % UNR)

            @pl.when(g + 2 < NCHT)
            def _issue_gather():
                @pl.when(g >= 3)
                def _drain():
                    scatter(bi2, br2).wait()
                idx_wait(g + 2, bi2)
                gather(g + 2, bi2, br2).start()

            gather(g, u % UNR, b5).wait()

            def scale(j16, _):
                evec = ewbufs[u % UNR, pl.ds(j16 * 16, 16)]
                for j in range(16):
                    sc = evec[j]
                    k = j16 * 16 + j
                    for d in range(HD // 16):
                        sl = pl.ds(d * 16, 16)
                        rows[b5, k, sl] = rows[b5, k, sl] * sc
                return 0

            lax.fori_loop(0, K // 16, scale, 0)
            pltpu.async_copy(rows.at[b5], acc.at[ebufs.at[u % UNR, 1]],
                             ssem.at[b5], add=True)
        return 0

    lax.fori_loop(0, NCHT // UNR, round_, 0)
    for b in range(NRB):
        scatter(0, b).wait()
    plsc.subcore_barrier()
    pltpu.sync_copy(acc.at[pl.ds(s * npr, npr)],
                    out_hbm.at[c, pl.ds(s * npr, npr)])


_RB = 1000


def _dinv_body(dref, dinv_ref):
    v = dref[...]
    deg = 1.0 + v[:NPAD // D] + v[NPAD // D:]
    dinv_ref[...] = jnp.where(deg > 0, lax.rsqrt(jnp.maximum(deg, 1e-30)),
                              0.0)


def _mm_body(xb, wb, di, ob):
    y = di[...] * jnp.dot(xb[...], wb[...],
                          preferred_element_type=jnp.float32)
    ob[0] = y[:, :HD]
    ob[1] = y[:, HD:]


def _merge_mm_body(p0, p1, hl, hr, di, bb, wb, ob):
    t = di[...] * (jnp.concatenate((p0[0], p1[0]), axis=1)
                   + jnp.concatenate((hl[0], hr[0]), axis=1)) + bb[...]
    a = jnp.where(t > 0, t, (jnp.exp(t) - 1.0))
    y = di[...] * jnp.dot(a, wb[...], preferred_element_type=jnp.float32)
    ob[0] = y[:, :HD]
    ob[1] = y[:, HD:]


def _final_body(p0, p1, hl, hr, di, bb, wb, blb, ob):
    t = di[...] * (jnp.concatenate((p0[0], p1[0]), axis=1)
                   + jnp.concatenate((hl[0], hr[0]), axis=1)) + bb[...]
    a = jnp.where(t > 0, t, (jnp.exp(t) - 1.0))
    y = jnp.dot(a, wb[...], preferred_element_type=jnp.float32) + blb[...]
    ob[...] = jnp.maximum(y, 0.0)


def _mm(x, W, di):
    return pl.pallas_call(
        _mm_body,
        grid=(N // _RB,),
        in_specs=[pl.BlockSpec((_RB, D), lambda i: (i, 0)),
                  pl.BlockSpec((D, D), lambda i: (0, 0)),
                  pl.BlockSpec((_RB, 1), lambda i: (i, 0))],
        out_specs=pl.BlockSpec((2, _RB, HD), lambda i: (0, i, 0)),
        out_shape=jax.ShapeDtypeStruct((2, N, HD), jnp.float32),
    )(x, W, di)


def _merge_mm(p, hst, di, b, W):
    return pl.pallas_call(
        _merge_mm_body,
        grid=(N // _RB,),
        in_specs=[pl.BlockSpec((1, _RB, HD), lambda i: (0, i, 0)),
                  pl.BlockSpec((1, _RB, HD), lambda i: (1, i, 0)),
                  pl.BlockSpec((1, _RB, HD), lambda i: (0, i, 0)),
                  pl.BlockSpec((1, _RB, HD), lambda i: (1, i, 0)),
                  pl.BlockSpec((_RB, 1), lambda i: (i, 0)),
                  pl.BlockSpec((1, D), lambda i: (0, 0)),
                  pl.BlockSpec((D, D), lambda i: (0, 0))],
        out_specs=pl.BlockSpec((2, _RB, HD), lambda i: (0, i, 0)),
        out_shape=jax.ShapeDtypeStruct((2, N, HD), jnp.float32),
    )(p, p, hst, hst, di, b, W)


def _final(p, hst, di, b, W, bl):
    return pl.pallas_call(
        _final_body,
        grid=(N // _RB,),
        in_specs=[pl.BlockSpec((1, _RB, HD), lambda i: (0, i, 0)),
                  pl.BlockSpec((1, _RB, HD), lambda i: (1, i, 0)),
                  pl.BlockSpec((1, _RB, HD), lambda i: (0, i, 0)),
                  pl.BlockSpec((1, _RB, HD), lambda i: (1, i, 0)),
                  pl.BlockSpec((_RB, 1), lambda i: (i, 0)),
                  pl.BlockSpec((1, D), lambda i: (0, 0)),
                  pl.BlockSpec((D, D), lambda i: (0, 0)),
                  pl.BlockSpec((1, D), lambda i: (0, 0))],
        out_specs=pl.BlockSpec((_RB, D), lambda i: (i, 0)),
        out_shape=jax.ShapeDtypeStruct((N, D), jnp.float32),
    )(p, p, hst, hst, di, b, W, bl)


def _dinv_call(degp):
    return pl.pallas_call(
        _dinv_body,
        out_shape=jax.ShapeDtypeStruct((NPAD // D, D), jnp.float32),
    )(degp)


def kernel(x, edge_index, edge_feats, W1, b1, W2, b2, W3, b3, Wlin, blin):
    row = edge_index[0]
    col = edge_index[1]
    ew = edge_feats

    col2 = col.reshape(NW, NCH, K)
    ew2 = ew.reshape(NW, NCH, K)
    edata = jnp.stack(
        [row.reshape(16, NCHT, K), col.reshape(16, NCHT, K)],
        axis=2)
    ew3 = ew.reshape(16, NCHT, K)

    degp = _deg(col2, ew2)
    dinv2d = _dinv_call(degp.reshape(2 * NPAD // D, D))
    di = dinv2d.reshape(NPAD)[:N].reshape(N, 1)

    b1r, b2r, b3r = b1.reshape(1, D), b2.reshape(1, D), b3.reshape(1, D)
    blr = blin.reshape(1, D)

    hst1 = _mm(x, W1, di)
    p = _spmm(hst1, edata, ew3)
    hst2 = _merge_mm(p, hst1, di, b1r, W2)
    p = _spmm(hst2, edata, ew3)
    hst3 = _merge_mm(p, hst2, di, b2r, W3)
    p = _spmm(hst3, edata, ew3)
    return _final(p, hst3, di, b3r, Wlin, blr)


# speedup vs baseline: 22.9818x; 1.9783x over previous
import functools

import jax
import jax.numpy as jnp
from jax import lax
from jax.experimental import pallas as pl
from jax.experimental.pallas import tpu as pltpu
from jax.experimental.pallas import tpu_sc as plsc

N = 10000
E = 320000
D = 128
HD = D // 2
NPAD = 10240
NW = 32
K = 80
NCH = E // NW // K
NB = 5
EPT = E // 16
NCHT = EPT // K
UNR = 10
NRB = 5

_mesh = plsc.VectorSubcoreMesh(core_axis_name="c", subcore_axis_name="s")


def _wid():
    return lax.axis_index("s") * 2 + lax.axis_index("c")


@functools.partial(
    pl.kernel, mesh=_mesh,
    out_type=jax.ShapeDtypeStruct((2, NPAD), jnp.float32),
    scratch_types=[
        pltpu.VMEM((NCH, K), jnp.int32),
        pltpu.VMEM((NCH, K), jnp.float32),
        pltpu.VMEM((NPAD // 16,), jnp.float32),
        pltpu.VMEM_SHARED((NPAD,), jnp.float32),
        pltpu.SemaphoreType.DMA,
    ],
)
def _deg(col2_hbm, ew2_hbm, out_hbm, cv2, ewv2, zv, acc, sem):
    c = lax.axis_index("c")
    s = lax.axis_index("s")
    nz = NPAD // 16

    def zfill(r, _):
        zv[pl.ds(r * 16, 16)] = jnp.zeros((16,), jnp.float32)
        return 0

    lax.fori_loop(0, nz // 16, zfill, 0)
    pltpu.sync_copy(zv, acc.at[pl.ds(s * nz, nz)])
    w = _wid()
    pltpu.sync_copy(col2_hbm.at[w], cv2)
    pltpu.sync_copy(ew2_hbm.at[w], ewv2)
    plsc.subcore_barrier()

    def round_(r, _):
        for b in range(NB):
            g = r * NB + b
            pltpu.async_copy(ewv2.at[g], acc.at[cv2.at[g]], sem, add=True)
        for b in range(NB):
            g = r * NB + b
            pltpu.make_async_copy(ewv2.at[g], acc.at[cv2.at[g]], sem).wait()
        return 0

    lax.fori_loop(0, NCH // NB, round_, 0)
    plsc.subcore_barrier()
    pltpu.sync_copy(acc.at[pl.ds(s * nz, nz)], out_hbm.at[c, pl.ds(s * nz, nz)])


@functools.partial(
    pl.kernel, mesh=_mesh,
    compiler_params=pltpu.CompilerParams(use_tc_tiling_on_sc=False),
    out_type=jax.ShapeDtypeStruct((2, NPAD, HD), jnp.float32),
    scratch_types=[
        pltpu.VMEM((UNR, 2, K), jnp.int32),
        pltpu.VMEM((UNR, K), jnp.float32),
        pltpu.VMEM((NRB, K, HD), jnp.float32),
        pltpu.VMEM_SHARED((NPAD, HD), jnp.float32),
        pltpu.SemaphoreType.DMA((UNR,)),
        pltpu.SemaphoreType.DMA((NRB,)),
        pltpu.SemaphoreType.DMA((NRB,)),
    ],
)
def _spmm(hst_hbm, edata_hbm, ew3_hbm, out_hbm, ebufs, ewbufs, rows, acc,
          isem, gsem, ssem):
    c = lax.axis_index("c")
    s = lax.axis_index("s")
    npr = NPAD // 16

    def zfill(r, _):
        for d in range(HD // 16):
            rows[0, r, pl.ds(d * 16, 16)] = jnp.zeros((16,), jnp.float32)
        return 0

    lax.fori_loop(0, K, zfill, 0)
    for j in range(npr // K):
        pltpu.sync_copy(rows.at[0], acc.at[pl.ds(s * npr + j * K, K)])
    plsc.subcore_barrier()

    def idx_load(g, slot):
        return (pltpu.make_async_copy(edata_hbm.at[s, g], ebufs.at[slot],
                                      isem.at[slot]),
                pltpu.make_async_copy(ew3_hbm.at[s, g], ewbufs.at[slot],
                                      isem.at[slot]))

    def idx_start(g, slot):
        a, b = idx_load(g, slot)
        a.start()
        b.start()

    def idx_wait(g, slot):
        a, b = idx_load(g, slot)
        a.wait()
        b.wait()

    def gather(g, slot, rslot):
        src = hst_hbm.at[c].at[ebufs.at[slot, 0]]
        return pltpu.make_async_copy(src, rows.at[rslot], gsem.at[rslot])

    def scatter(slot, rslot):
        return pltpu.make_async_copy(rows.at[rslot],
                                     acc.at[ebufs.at[slot, 1]],
                                     ssem.at[rslot])

    for g in range(4):
        idx_start(g, g)
    for g in range(2):
        idx_wait(g, g)
        gather(g, g, g).start()

    def round_(r, _):
        for u in range(UNR):
            g = r * UNR + u
            b5 = u % NRB
            bi2 = (u + 2) % UNR
            br2 = (u + 2) % NRB

            @pl.when(g + 4 < NCHT)
            def _issue_idx():
                idx_start(g + 4, (u + 4) % UNR)

            @pl.when(g + 2 < NCHT)
            def _issue_gather():
                @pl.when(g >= 3)
                def _drain():
                    scatter(bi2, br2).wait()
                idx_wait(g + 2, bi2)
                gather(g + 2, bi2, br2).start()

            gather(g, u % UNR, b5).wait()

            def scale(j16, _):
                evec = ewbufs[u % UNR, pl.ds(j16 * 16, 16)]
                for j in range(0, 16, 2):
                    k0 = j16 * 16 + j
                    k1 = k0 + 1
                    sc0 = evec[j]
                    sc1 = evec[j + 1]
                    v0 = [rows[b5, k0, pl.ds(d * 16, 16)]
                          for d in range(HD // 16)]
                    v1 = [rows[b5, k1, pl.ds(d * 16, 16)]
                          for d in range(HD // 16)]
                    for d in range(HD // 16):
                        rows[b5, k0, pl.ds(d * 16, 16)] = v0[d] * sc0
                    for d in range(HD // 16):
                        rows[b5, k1, pl.ds(d * 16, 16)] = v1[d] * sc1
                return 0

            lax.fori_loop(0, K // 16, scale, 0)
            pltpu.async_copy(rows.at[b5], acc.at[ebufs.at[u % UNR, 1]],
                             ssem.at[b5], add=True)
        return 0

    lax.fori_loop(0, NCHT // UNR, round_, 0)
    for b in range(NRB):
        scatter(0, b).wait()
    plsc.subcore_barrier()
    pltpu.sync_copy(acc.at[pl.ds(s * npr, npr)],
                    out_hbm.at[c, pl.ds(s * npr, npr)])


_RB = 1000


def _dinv_body(dref, dinv_ref):
    v = dref[...]
    deg = 1.0 + v[:NPAD // D] + v[NPAD // D:]
    dinv_ref[...] = jnp.where(deg > 0, lax.rsqrt(jnp.maximum(deg, 1e-30)),
                              0.0)


def _mm_body(xb, wb, di, ob):
    y = di[...] * jnp.dot(xb[...], wb[...],
                          preferred_element_type=jnp.float32)
    ob[0] = y[:, :HD]
    ob[1] = y[:, HD:]


def _merge_mm_body(p0, p1, hl, hr, di, bb, wb, ob):
    t = di[...] * (jnp.concatenate((p0[0], p1[0]), axis=1)
                   + jnp.concatenate((hl[0], hr[0]), axis=1)) + bb[...]
    a = jnp.where(t > 0, t, (jnp.exp(t) - 1.0))
    y = di[...] * jnp.dot(a, wb[...], preferred_element_type=jnp.float32)
    ob[0] = y[:, :HD]
    ob[1] = y[:, HD:]


def _final_body(p0, p1, hl, hr, di, bb, wb, blb, ob):
    t = di[...] * (jnp.concatenate((p0[0], p1[0]), axis=1)
                   + jnp.concatenate((hl[0], hr[0]), axis=1)) + bb[...]
    a = jnp.where(t > 0, t, (jnp.exp(t) - 1.0))
    y = jnp.dot(a, wb[...], preferred_element_type=jnp.float32) + blb[...]
    ob[...] = jnp.maximum(y, 0.0)


def _mm(x, W, di):
    return pl.pallas_call(
        _mm_body,
        grid=(N // _RB,),
        in_specs=[pl.BlockSpec((_RB, D), lambda i: (i, 0)),
                  pl.BlockSpec((D, D), lambda i: (0, 0)),
                  pl.BlockSpec((_RB, 1), lambda i: (i, 0))],
        out_specs=pl.BlockSpec((2, _RB, HD), lambda i: (0, i, 0)),
        out_shape=jax.ShapeDtypeStruct((2, N, HD), jnp.float32),
    )(x, W, di)


def _merge_mm(p, hst, di, b, W):
    return pl.pallas_call(
        _merge_mm_body,
        grid=(N // _RB,),
        in_specs=[pl.BlockSpec((1, _RB, HD), lambda i: (0, i, 0)),
                  pl.BlockSpec((1, _RB, HD), lambda i: (1, i, 0)),
                  pl.BlockSpec((1, _RB, HD), lambda i: (0, i, 0)),
                  pl.BlockSpec((1, _RB, HD), lambda i: (1, i, 0)),
                  pl.BlockSpec((_RB, 1), lambda i: (i, 0)),
                  pl.BlockSpec((1, D), lambda i: (0, 0)),
                  pl.BlockSpec((D, D), lambda i: (0, 0))],
        out_specs=pl.BlockSpec((2, _RB, HD), lambda i: (0, i, 0)),
        out_shape=jax.ShapeDtypeStruct((2, N, HD), jnp.float32),
    )(p, p, hst, hst, di, b, W)


def _final(p, hst, di, b, W, bl):
    return pl.pallas_call(
        _final_body,
        grid=(N // _RB,),
        in_specs=[pl.BlockSpec((1, _RB, HD), lambda i: (0, i, 0)),
                  pl.BlockSpec((1, _RB, HD), lambda i: (1, i, 0)),
                  pl.BlockSpec((1, _RB, HD), lambda i: (0, i, 0)),
                  pl.BlockSpec((1, _RB, HD), lambda i: (1, i, 0)),
                  pl.BlockSpec((_RB, 1), lambda i: (i, 0)),
                  pl.BlockSpec((1, D), lambda i: (0, 0)),
                  pl.BlockSpec((D, D), lambda i: (0, 0)),
                  pl.BlockSpec((1, D), lambda i: (0, 0))],
        out_specs=pl.BlockSpec((_RB, D), lambda i: (i, 0)),
        out_shape=jax.ShapeDtypeStruct((N, D), jnp.float32),
    )(p, p, hst, hst, di, b, W, bl)


def _dinv_call(degp):
    return pl.pallas_call(
        _dinv_body,
        out_shape=jax.ShapeDtypeStruct((NPAD // D, D), jnp.float32),
    )(degp)


def kernel(x, edge_index, edge_feats, W1, b1, W2, b2, W3, b3, Wlin, blin):
    row = edge_index[0]
    col = edge_index[1]
    ew = edge_feats

    col2 = col.reshape(NW, NCH, K)
    ew2 = ew.reshape(NW, NCH, K)
    edata = jnp.stack(
        [row.reshape(16, NCHT, K), col.reshape(16, NCHT, K)],
        axis=2)
    ew3 = ew.reshape(16, NCHT, K)

    degp = _deg(col2, ew2)
    dinv2d = _dinv_call(degp.reshape(2 * NPAD // D, D))
    di = dinv2d.reshape(NPAD)[:N].reshape(N, 1)

    b1r, b2r, b3r = b1.reshape(1, D), b2.reshape(1, D), b3.reshape(1, D)
    blr = blin.reshape(1, D)

    hst1 = _mm(x, W1, di)
    p = _spmm(hst1, edata, ew3)
    hst2 = _merge_mm(p, hst1, di, b1r, W2)
    p = _spmm(hst2, edata, ew3)
    hst3 = _merge_mm(p, hst2, di, b2r, W3)
    p = _spmm(hst3, edata, ew3)
    return _final(p, hst3, di, b3r, Wlin, blr)
